# Initial kernel scaffold; baseline (speedup 1.0000x reference)
import jax
import jax.numpy as jnp
from jax.experimental import pallas as pl


def kernel(node_features, edge_index, batch_index, W1, b1, W2, b2):
    raise NotImplementedError("write your pallas kernel here")


# speedup vs baseline: 9.9780x; 9.9780x over previous
import functools

import jax
import jax.numpy as jnp
from jax import lax
from jax.experimental import pallas as pl
from jax.experimental.pallas import tpu as pltpu
from jax.experimental.pallas import tpu_sc as plsc

D = 128
G = 16
NC = 2
NS = 16
NW = NC * NS
EB = 128
BLK = 512


def _pad_sizes(n_nodes, n_edges):
    npad = ((n_nodes + 1 + BLK - 1) // BLK) * BLK
    chunk = NW * EB
    epad = ((n_edges + chunk - 1) // chunk) * chunk
    return npad, epad


def _deg_kernel(np_, nbatch):
    rows_s = np_ // NS
    mesh = plsc.VectorSubcoreMesh(core_axis_name="c", subcore_axis_name="s")

    @functools.partial(
        pl.kernel,
        out_type=jax.ShapeDtypeStruct((NC, np_, D), jnp.float32),
        mesh=mesh,
        scratch_types=[
            pltpu.VMEM((EB,), jnp.int32),
            pltpu.VMEM((EB, D), jnp.float32),
            pltpu.VMEM_SHARED((np_, D), jnp.float32),
        ],
    )
    def k(dst_hbm, ones_hbm, zeros_hbm, out_hbm, didx, onesb, acc):
        c = lax.axis_index("c")
        s = lax.axis_index("s")
        wid = s * NC + c
        pltpu.sync_copy(zeros_hbm.at[pl.ds(s * rows_s, rows_s)],
                        acc.at[pl.ds(s * rows_s, rows_s)])
        pltpu.sync_copy(ones_hbm, onesb)
        plsc.subcore_barrier()
        base = wid * (nbatch * EB)

        def body(i, carry):
            pltpu.sync_copy(dst_hbm.at[pl.ds(base + i * EB, EB)], didx)
            pltpu.sync_copy(onesb, acc.at[didx], add=True)
            return carry

        lax.fori_loop(0, nbatch, body, 0)
        plsc.subcore_barrier()
        pltpu.sync_copy(acc.at[pl.ds(s * rows_s, rows_s)],
                        out_hbm.at[c, pl.ds(s * rows_s, rows_s)])

    return k


def _agg_kernel(np_, nbatch):
    rows_s = np_ // NS
    mesh = plsc.VectorSubcoreMesh(core_axis_name="c", subcore_axis_name="s")

    @functools.partial(
        pl.kernel,
        out_type=jax.ShapeDtypeStruct((NC, np_, D), jnp.float32),
        mesh=mesh,
        scratch_types=[
            pltpu.VMEM((EB,), jnp.int32),
            pltpu.VMEM((EB,), jnp.int32),
            pltpu.VMEM((EB, D), jnp.float32),
            pltpu.VMEM_SHARED((np_, D), jnp.float32),
            pltpu.SemaphoreType.DMA,
        ],
    )
    def k(h_hbm, src_hbm, dst_hbm, zeros_hbm, out_hbm, sidx, didx, rows, acc, sem):
        c = lax.axis_index("c")
        s = lax.axis_index("s")
        wid = s * NC + c
        pltpu.sync_copy(zeros_hbm.at[pl.ds(s * rows_s, rows_s)],
                        acc.at[pl.ds(s * rows_s, rows_s)])
        plsc.subcore_barrier()
        base = wid * (nbatch * EB)

        def body(i, carry):
            off = base + i * EB
            pltpu.sync_copy(src_hbm.at[pl.ds(off, EB)], sidx)
            pltpu.sync_copy(dst_hbm.at[pl.ds(off, EB)], didx)
            pltpu.async_copy(h_hbm.at[sidx], rows, sem).wait()
            pltpu.sync_copy(rows, acc.at[didx], add=True)
            return carry

        lax.fori_loop(0, nbatch, body, 0)
        plsc.subcore_barrier()
        pltpu.sync_copy(acc.at[pl.ds(s * rows_s, rows_s)],
                        out_hbm.at[c, pl.ds(s * rows_s, rows_s)])

    return k


def _dis(dg_ref):
    deg = dg_ref[0, :, 0:1] + dg_ref[1, :, 0:1] + 1.0
    return lax.rsqrt(deg)


def _scale_matmul_call(np_, xp, w1, degacc):
    def body(x_ref, w_ref, dg_ref, out_ref):
        dis = _dis(dg_ref)
        h = jnp.dot(x_ref[...], w_ref[...],
                    preferred_element_type=jnp.float32,
                    precision=lax.Precision.HIGHEST)
        out_ref[...] = dis * h

    return pl.pallas_call(
        body,
        grid=(np_ // BLK,),
        in_specs=[
            pl.BlockSpec((BLK, D), lambda i: (i, 0)),
            pl.BlockSpec((D, D), lambda i: (0, 0)),
            pl.BlockSpec((NC, BLK, D), lambda i: (0, i, 0)),
        ],
        out_specs=pl.BlockSpec((BLK, D), lambda i: (i, 0)),
        out_shape=jax.ShapeDtypeStruct((np_, D), jnp.float32),
    )(xp, w1, degacc)


def _mid_layer_call(np_, aggpair, hp, degacc, b, w2):
    def body(ag_ref, hp_ref, dg_ref, b_ref, w_ref, out_ref):
        dis = _dis(dg_ref)
        x1 = jnp.maximum(
            dis * (ag_ref[0] + ag_ref[1] + hp_ref[...]) + b_ref[...], 0.0)
        h2 = jnp.dot(x1, w_ref[...],
                     preferred_element_type=jnp.float32,
                     precision=lax.Precision.HIGHEST)
        out_ref[...] = dis * h2

    return pl.pallas_call(
        body,
        grid=(np_ // BLK,),
        in_specs=[
            pl.BlockSpec((NC, BLK, D), lambda i: (0, i, 0)),
            pl.BlockSpec((BLK, D), lambda i: (i, 0)),
            pl.BlockSpec((NC, BLK, D), lambda i: (0, i, 0)),
            pl.BlockSpec((1, D), lambda i: (0, 0)),
            pl.BlockSpec((D, D), lambda i: (0, 0)),
        ],
        out_specs=pl.BlockSpec((BLK, D), lambda i: (i, 0)),
        out_shape=jax.ShapeDtypeStruct((np_, D), jnp.float32),
    )(aggpair, hp, degacc, b, w2)


def _final_pool_call(np_, aggpair, hp, degacc, b, bidx):
    nblk = np_ // BLK

    def body(ag_ref, hp_ref, dg_ref, b_ref, bi_ref, out_ref, acc, cnt):
        i = pl.program_id(0)

        @pl.when(i == 0)
        def _init():
            acc[...] = jnp.zeros_like(acc)
            cnt[...] = jnp.zeros_like(cnt)

        dis = _dis(dg_ref)
        x2 = jnp.maximum(
            dis * (ag_ref[0] + ag_ref[1] + hp_ref[...]) + b_ref[...], 0.0)
        onehot = (bi_ref[...] ==
                  lax.broadcasted_iota(jnp.int32, (BLK, G), 1)
                  ).astype(jnp.float32)
        dn = (((0,), (0,)), ((), ()))
        acc[...] += lax.dot_general(onehot, x2, dn,
                                    preferred_element_type=jnp.float32,
                                    precision=lax.Precision.HIGHEST)
        cnt[...] += lax.dot_general(onehot, jnp.ones((BLK, D), jnp.float32),
                                    dn, preferred_element_type=jnp.float32,
                                    precision=lax.Precision.HIGHEST)

        @pl.when(i == nblk - 1)
        def _fin():
            out_ref[...] = acc[...] / jnp.maximum(cnt[...], 1.0)

    return pl.pallas_call(
        body,
        grid=(nblk,),
        in_specs=[
            pl.BlockSpec((NC, BLK, D), lambda i: (0, i, 0)),
            pl.BlockSpec((BLK, D), lambda i: (i, 0)),
            pl.BlockSpec((NC, BLK, D), lambda i: (0, i, 0)),
            pl.BlockSpec((1, D), lambda i: (0, 0)),
            pl.BlockSpec((BLK, 1), lambda i: (i, 0)),
        ],
        out_specs=pl.BlockSpec((G, D), lambda i: (0, 0)),
        out_shape=jax.ShapeDtypeStruct((G, D), jnp.float32),
        scratch_shapes=[
            pltpu.VMEM((G, D), jnp.float32),
            pltpu.VMEM((G, D), jnp.float32),
        ],
    )(aggpair, hp, degacc, b, bidx)


def kernel(node_features, edge_index, batch_index, W1, b1, W2, b2):
    n = node_features.shape[0]
    e = edge_index.shape[1]
    np_, ep = _pad_sizes(n, e)
    nbatch = ep // (NW * EB)

    pad_idx = jnp.full((ep - e,), n, jnp.int32)
    src = jnp.concatenate([edge_index[0], pad_idx])
    dst = jnp.concatenate([edge_index[1], pad_idx])
    xp = jnp.concatenate(
        [node_features, jnp.zeros((np_ - n, D), jnp.float32)])
    bidx = jnp.concatenate(
        [batch_index, jnp.full((np_ - n,), G, jnp.int32)]).reshape(np_, 1)
    zeros_nd = jnp.zeros((np_, D), jnp.float32)
    ones_eb = jnp.ones((EB, D), jnp.float32)

    deg_k = _deg_kernel(np_, nbatch)
    agg_k = _agg_kernel(np_, nbatch)

    degacc = deg_k(dst, ones_eb, zeros_nd)
    h1p = _scale_matmul_call(np_, xp, W1, degacc)
    agg1 = agg_k(h1p, src, dst, zeros_nd)
    h2p = _mid_layer_call(np_, agg1, h1p, degacc, b1.reshape(1, D), W2)
    agg2 = agg_k(h2p, src, dst, zeros_nd)
    return _final_pool_call(np_, agg2, h2p, degacc, b2.reshape(1, D), bidx)
